# baseline (speedup 1.0000x reference)
import functools

import jax
import jax.numpy as jnp
from jax import lax
from jax.experimental import pallas as pl
from jax.experimental.pallas import tpu as pltpu
from jax.experimental.pallas import tpu_sc as plsc

_NUM_CORES = 2
_NUM_SUBCORES = 16
_GE = 32


def kernel(board_tensor, emb_table):
    B, S, F = board_tensor.shape
    SN, E = emb_table.shape
    NW = _NUM_CORES * _NUM_SUBCORES
    bpw = B // NW
    n_emb = bpw // _GE

    mesh = plsc.VectorSubcoreMesh(
        core_axis_name="c", subcore_axis_name="s", num_cores=_NUM_CORES
    )

    @functools.partial(
        pl.kernel,
        out_type=jax.ShapeDtypeStruct((B, S, F + E), jnp.float32),
        mesh=mesh,
        compiler_params=pltpu.CompilerParams(use_tc_tiling_on_sc=False),
        scratch_types=[
            pltpu.VMEM((_GE, SN, E), jnp.float32),
            pltpu.SemaphoreType.DMA,
            pltpu.SemaphoreType.DMA,
            pltpu.SemaphoreType.DMA,
        ],
    )
    def k(board_hbm, emb_hbm, out_hbm, rep_v, sem_stage, sem_board, sem_emb):
        wid = lax.axis_index("s") * _NUM_CORES + lax.axis_index("c")
        base = wid * bpw

        for i in range(_GE):
            pltpu.make_async_copy(emb_hbm, rep_v.at[i], sem_stage).start()

        board_cp = pltpu.make_async_copy(
            board_hbm.at[pl.ds(base, bpw)],
            out_hbm.at[pl.ds(base, bpw), :, pl.ds(0, F)],
            sem_board,
        )
        board_cp.start()

        for i in range(_GE):
            pltpu.make_async_copy(emb_hbm, rep_v.at[i], sem_stage).wait()

        emb_cps = []
        for i in range(n_emb):
            cp = pltpu.make_async_copy(
                rep_v,
                out_hbm.at[pl.ds(base + i * _GE, _GE), :, pl.ds(F, E)],
                sem_emb,
            )
            cp.start()
            emb_cps.append(cp)

        board_cp.wait()
        for cp in emb_cps:
            cp.wait()

    return k(board_tensor, emb_table)
